# Initial kernel scaffold; baseline (speedup 1.0000x reference)
import jax
import jax.numpy as jnp
from jax.experimental import pallas as pl


def kernel(x, edge_index, W_l, b_l, W_r, gamma, beta):
    raise NotImplementedError("write your pallas kernel here")


# speedup vs baseline: 5.2742x; 5.2742x over previous
import jax
import jax.numpy as jnp
from jax import lax
from jax.experimental import pallas as pl
from jax.experimental.pallas import tpu as pltpu
from jax.experimental.pallas import tpu_sc as plsc

_NC = 2
_NS = 16
_B = 128
_R = 400


def _round_up(v, m):
  return (v + m - 1) // m * m


def _sage_aggregate(x, src, dst, n_pad):
  n, d = x.shape
  e = src.shape[0]
  nw = _NC * _NS
  batches_per_tile = e // (nw * _B)
  rows_per_tile = n_pad // _NS
  assert e % (nw * _B) == 0
  assert rows_per_tile % _B == 0

  mesh = plsc.VectorSubcoreMesh(core_axis_name="c", subcore_axis_name="s")

  def body(x_hbm, src_hbm, dst_hbm, sum_hbm, cnt_hbm,
           idx_s, idx_d, rows_v, ones_v, zrow_v, acc_sh, cnt_sh):
    c = lax.axis_index("c")
    s = lax.axis_index("s")
    w = c * _NS + s
    row0 = s * rows_per_tile

    @pl.loop(0, _B)
    def _(r):
      @pl.loop(0, d, step=16)
      def _(k):
        rows_v[r, pl.ds(k, 16)] = jnp.zeros((16,), jnp.float32)

    @pl.loop(0, rows_per_tile, step=16)
    def _(i):
      zrow_v[pl.ds(i, 16)] = jnp.zeros((16,), jnp.float32)

    @pl.loop(0, _B, step=16)
    def _(i):
      ones_v[pl.ds(i, 16)] = jnp.ones((16,), jnp.float32)

    for j in range(rows_per_tile // _B):
      pltpu.sync_copy(rows_v, acc_sh.at[pl.ds(row0 + j * _B, _B)])
    pltpu.sync_copy(zrow_v, cnt_sh.at[pl.ds(row0, rows_per_tile)])
    plsc.subcore_barrier()

    base = w * batches_per_tile * _B

    @pl.loop(0, batches_per_tile)
    def _(b):
      off = base + b * _B
      pltpu.sync_copy(src_hbm.at[pl.ds(off, _B)], idx_s.at[0])
      pltpu.sync_copy(dst_hbm.at[pl.ds(off, _B)], idx_d.at[0])
      pltpu.sync_copy(x_hbm.at[idx_s.at[0]], rows_v)
      pltpu.sync_copy(rows_v, acc_sh.at[idx_d.at[0]], add=True)
      pltpu.sync_copy(ones_v, cnt_sh.at[idx_d.at[0]], add=True)

    plsc.subcore_barrier()

    pltpu.sync_copy(acc_sh.at[pl.ds(row0, rows_per_tile)],
                    sum_hbm.at[c, pl.ds(row0, rows_per_tile)])
    pltpu.sync_copy(cnt_sh.at[pl.ds(row0, rows_per_tile)],
                    cnt_hbm.at[pl.ds(c * n_pad + row0, rows_per_tile)])

  kern = pl.kernel(
      body,
      out_type=[
          jax.ShapeDtypeStruct((_NC, n_pad, d), jnp.float32),
          jax.ShapeDtypeStruct((_NC * n_pad,), jnp.float32),
      ],
      mesh=mesh,
      scratch_types=[
          pltpu.VMEM((1, _B), jnp.int32),
          pltpu.VMEM((1, _B), jnp.int32),
          pltpu.VMEM((_B, d), jnp.float32),
          pltpu.VMEM((_B,), jnp.float32),
          pltpu.VMEM((rows_per_tile,), jnp.float32),
          pltpu.VMEM_SHARED((n_pad, d), jnp.float32),
          pltpu.VMEM_SHARED((n_pad,), jnp.float32),
      ],
  )
  return kern(x, src, dst)


def _dense_body(sum_ref, cnt_ref, x_ref, wl_ref, bl_ref, wr_ref, g_ref, b_ref,
                o_ref):
  s = sum_ref[0] + sum_ref[1]
  c = cnt_ref[0] + cnt_ref[1]
  aggr = s / jnp.maximum(c, 1.0)
  xb = x_ref[...]
  f = (lax.dot_general(aggr, wl_ref[...], (((1,), (1,)), ((), ())),
                       preferred_element_type=jnp.float32)
       + lax.dot_general(xb, wr_ref[...], (((1,), (1,)), ((), ())),
                         preferred_element_type=jnp.float32)
       + bl_ref[...])
  f = 0.5 * f * (1.0 + lax.erf(f * (2.0 ** -0.5)))
  mu = jnp.mean(f, axis=-1, keepdims=True)
  zc = f - mu
  var = jnp.mean(zc * zc, axis=-1, keepdims=True)
  o_ref[...] = zc * lax.rsqrt(var + 1e-5) * g_ref[...] + b_ref[...] + xb


def _dense(sums, cnt3, x, W_l, b_l, W_r, gamma, beta):
  n, d = x.shape
  grid = (n // _R,)
  return pl.pallas_call(
      _dense_body,
      grid=grid,
      in_specs=[
          pl.BlockSpec((_NC, _R, d), lambda i: (0, i, 0)),
          pl.BlockSpec((_NC, _R, 1), lambda i: (0, i, 0)),
          pl.BlockSpec((_R, d), lambda i: (i, 0)),
          pl.BlockSpec((d, d), lambda i: (0, 0)),
          pl.BlockSpec((1, d), lambda i: (0, 0)),
          pl.BlockSpec((d, d), lambda i: (0, 0)),
          pl.BlockSpec((1, d), lambda i: (0, 0)),
          pl.BlockSpec((1, d), lambda i: (0, 0)),
      ],
      out_specs=pl.BlockSpec((_R, d), lambda i: (i, 0)),
      out_shape=jax.ShapeDtypeStruct((n, d), jnp.float32),
  )(sums, cnt3, x, W_l, b_l.reshape(1, d), W_r, gamma.reshape(1, d),
    beta.reshape(1, d))


def kernel(x, edge_index, W_l, b_l, W_r, gamma, beta):
  n, d = x.shape
  e = edge_index.shape[1]
  n_pad = _round_up(n + 1, _NS * _B)
  e_pad = _round_up(e, _NC * _NS * _B)
  pad = e_pad - e
  src = jnp.concatenate([edge_index[0], jnp.zeros((pad,), jnp.int32)])
  dst = jnp.concatenate(
      [edge_index[1], jnp.full((pad,), n_pad - 1, jnp.int32)])
  sums, cnts = _sage_aggregate(x, src, dst, n_pad)
  cnt3 = cnts.reshape(_NC, n_pad, 1)
  return _dense(sums, cnt3, x, W_l, b_l, W_r, gamma, beta)
